# baseline (speedup 1.0000x reference)
import jax
import jax.numpy as jnp
from jax import lax
from jax.experimental import pallas as pl
from jax.experimental.pallas import tpu as pltpu
from jax.experimental.pallas import tpu_sc as plsc

_N = 10000
_E = 320000
_D = 128
_H = 8
_C = 16
_NC = 2
_NS = 16
_NW = _NC * _NS
_PERW = _E // _NW
_EPT = _E // _NS
_CH = 80
_GCH = _PERW // _CH
_SCH = _EPT // _CH
_NPAD = 10240
_RPT = _NPAD // _NS
_ZCH = 32

_BN = 1000
_BE = 2000

_F32 = jnp.float32

_sc_mesh = plsc.VectorSubcoreMesh(
    core_axis_name="c", subcore_axis_name="s", num_cores=_NC, num_subcores=_NS)


def _head_indicator(rows, cols, row_div, col_div):
    r = lax.broadcasted_iota(jnp.int32, (rows, cols), 0) // row_div
    c = lax.broadcasted_iota(jnp.int32, (rows, cols), 1) // col_div
    return (r == c).astype(_F32)


def _proj_body(x_ref, wq_ref, bq_ref, wkv_ref, bkv_ref, ws_ref, bs_ref,
               qn_ref, kv_ref, sk_ref):
    xb = x_ref[...]
    qn_ref[...] = jnp.dot(xb, wq_ref[...], preferred_element_type=_F32) + bq_ref[...]
    kv_ref[...] = jnp.dot(xb, wkv_ref[...], preferred_element_type=_F32) + bkv_ref[...]
    sk_ref[...] = jnp.dot(xb, ws_ref[...], preferred_element_type=_F32) + bs_ref[...]


_proj = pl.pallas_call(
    _proj_body,
    grid=(_N // _BN,),
    in_specs=[
        pl.BlockSpec((_BN, _D), lambda i: (i, 0)),
        pl.BlockSpec((_D, _D), lambda i: (0, 0)),
        pl.BlockSpec((1, _D), lambda i: (0, 0)),
        pl.BlockSpec((_D, 2 * _D), lambda i: (0, 0)),
        pl.BlockSpec((1, 2 * _D), lambda i: (0, 0)),
        pl.BlockSpec((_D, _D), lambda i: (0, 0)),
        pl.BlockSpec((1, _D), lambda i: (0, 0)),
    ],
    out_specs=[
        pl.BlockSpec((_BN, _D), lambda i: (i, 0)),
        pl.BlockSpec((_BN, 2 * _D), lambda i: (i, 0)),
        pl.BlockSpec((_BN, _D), lambda i: (i, 0)),
    ],
    out_shape=[
        jax.ShapeDtypeStruct((_N, _D), _F32),
        jax.ShapeDtypeStruct((_N, 2 * _D), _F32),
        jax.ShapeDtypeStruct((_N, _D), _F32),
    ],
)


def _sc_gather_body(qn_hbm, kv_hbm, dst_hbm, src_hbm, qd_hbm, kvs_hbm,
                    dstv, srcv, qa, kva, qb, kvb,
                    gqa, gkva, gqb, gkvb, wqa, wkva, wqb, wkvb):
    wid = lax.axis_index("s") * _NC + lax.axis_index("c")
    base = wid * _PERW
    pltpu.sync_copy(dst_hbm.at[wid], dstv)
    pltpu.sync_copy(src_hbm.at[wid], srcv)

    def pair(t, carry):
        ja = 2 * t
        jb = 2 * t + 1
        offa = base + ja * _CH
        offb = base + jb * _CH
        ga1 = pltpu.async_copy(qn_hbm.at[dstv.at[ja]], qa, gqa)
        ga2 = pltpu.async_copy(kv_hbm.at[srcv.at[ja]], kva, gkva)
        gb1 = pltpu.async_copy(qn_hbm.at[dstv.at[jb]], qb, gqb)
        gb2 = pltpu.async_copy(kv_hbm.at[srcv.at[jb]], kvb, gkvb)
        ga1.wait()
        ga2.wait()
        wa1 = pltpu.async_copy(qa, qd_hbm.at[pl.ds(offa, _CH)], wqa)
        wa2 = pltpu.async_copy(kva, kvs_hbm.at[pl.ds(offa, _CH)], wkva)
        gb1.wait()
        gb2.wait()
        wb1 = pltpu.async_copy(qb, qd_hbm.at[pl.ds(offb, _CH)], wqb)
        wb2 = pltpu.async_copy(kvb, kvs_hbm.at[pl.ds(offb, _CH)], wkvb)
        wa1.wait()
        wa2.wait()
        wb1.wait()
        wb2.wait()
        return carry

    lax.fori_loop(0, _GCH // 2, pair, 0)

    jt = _GCH - 1
    offt = base + jt * _CH
    t1 = pltpu.async_copy(qn_hbm.at[dstv.at[jt]], qa, gqa)
    t2 = pltpu.async_copy(kv_hbm.at[srcv.at[jt]], kva, gkva)
    t1.wait()
    t2.wait()
    pltpu.sync_copy(qa, qd_hbm.at[pl.ds(offt, _CH)])
    pltpu.sync_copy(kva, kvs_hbm.at[pl.ds(offt, _CH)])


_sc_gather = pl.kernel(
    _sc_gather_body,
    out_type=(
        jax.ShapeDtypeStruct((_E, _D), _F32),
        jax.ShapeDtypeStruct((_E, 2 * _D), _F32),
    ),
    mesh=_sc_mesh,
    scratch_types=[
        pltpu.VMEM((_GCH, _CH), jnp.int32),
        pltpu.VMEM((_GCH, _CH), jnp.int32),
        pltpu.VMEM((_CH, _D), _F32),
        pltpu.VMEM((_CH, 2 * _D), _F32),
        pltpu.VMEM((_CH, _D), _F32),
        pltpu.VMEM((_CH, 2 * _D), _F32),
    ] + [pltpu.SemaphoreType.DMA] * 8,
)


def _edge_body(ea_ref, qd_ref, kvs_ref, we_ref, m_ref, ex_ref):
    e = jnp.dot(ea_ref[...], we_ref[...], preferred_element_type=_F32)
    kk = kvs_ref[:, :_D] + e
    vv = kvs_ref[:, _D:] + e
    s_fold = _head_indicator(_D, _H, _C, 1)
    s_bcast = _head_indicator(_H, _D, 1, _C)
    alpha = jnp.dot(qd_ref[...] * kk, s_fold, preferred_element_type=_F32) * 0.25
    exv = jnp.exp(alpha)
    exb = jnp.dot(exv, s_bcast, preferred_element_type=_F32)
    m_ref[...] = vv * exb
    s_pad = _head_indicator(_H, _C, 1, 1)
    ex_ref[...] = jnp.dot(exv, s_pad, preferred_element_type=_F32)


_edges = pl.pallas_call(
    _edge_body,
    grid=(_E // _BE,),
    in_specs=[
        pl.BlockSpec((_BE, _D), lambda i: (i, 0)),
        pl.BlockSpec((_BE, _D), lambda i: (i, 0)),
        pl.BlockSpec((_BE, 2 * _D), lambda i: (i, 0)),
        pl.BlockSpec((_D, _D), lambda i: (0, 0)),
    ],
    out_specs=[
        pl.BlockSpec((_BE, _D), lambda i: (i, 0)),
        pl.BlockSpec((_BE, _C), lambda i: (i, 0)),
    ],
    out_shape=[
        jax.ShapeDtypeStruct((_E, _D), _F32),
        jax.ShapeDtypeStruct((_E, _C), _F32),
    ],
)


def _bcast16(v, g):
    return jnp.broadcast_to(lax.slice_in_dim(v, g, g + 1, axis=0), (_C,))


def _sc_scatter_body(m_hbm, ex_hbm, dst_hbm, z_hbm, acc_hbm,
                     ia, ib, ea16, eb16, ra, rb, zb, acc_s,
                     sia, sib, sla, slb, ssa, ssb):
    c = lax.axis_index("c")
    s = lax.axis_index("s")
    base = s * _EPT

    pltpu.sync_copy(z_hbm, zb)

    def zstep(k, carry):
        r0 = s * _RPT + k * _ZCH
        pltpu.sync_copy(zb, acc_s.at[pl.ds(r0, _ZCH)])
        return carry

    lax.fori_loop(0, _RPT // _ZCH, zstep, 0)
    plsc.subcore_barrier()

    def _expand(src16, dst128):
        def erow(r, carry):
            v = src16[r]
            for g in range(_H):
                dst128[r, pl.ds(_C * g, _C)] = _bcast16(v, g)
            return carry

        lax.fori_loop(0, _CH, erow, 0)

    def pair(t, carry):
        offa = base + (2 * t) * _CH
        offb = base + (2 * t + 1) * _CH
        cia = pltpu.async_copy(dst_hbm.at[pl.ds(offa, _CH)], ia, sia)
        cib = pltpu.async_copy(dst_hbm.at[pl.ds(offb, _CH)], ib, sib)

        @pl.when(c == 0)
        def _load_m():
            pltpu.async_copy(m_hbm.at[pl.ds(offa, _CH)], ra, sla)
            pltpu.async_copy(m_hbm.at[pl.ds(offb, _CH)], rb, slb)
            pltpu.make_async_copy(m_hbm.at[pl.ds(offa, _CH)], ra, sla).wait()

        @pl.when(c == 1)
        def _load_ex():
            pltpu.async_copy(ex_hbm.at[pl.ds(offa, _CH)], ea16, sla)
            pltpu.async_copy(ex_hbm.at[pl.ds(offb, _CH)], eb16, slb)
            pltpu.make_async_copy(ex_hbm.at[pl.ds(offa, _CH)], ea16, sla).wait()
            _expand(ea16, ra)

        cia.wait()
        sa = pltpu.async_copy(ra, acc_s.at[ia], ssa, add=True)

        @pl.when(c == 0)
        def _wait_m_b():
            pltpu.make_async_copy(m_hbm.at[pl.ds(offb, _CH)], rb, slb).wait()

        @pl.when(c == 1)
        def _wait_ex_b():
            pltpu.make_async_copy(ex_hbm.at[pl.ds(offb, _CH)], eb16, slb).wait()
            _expand(eb16, rb)

        cib.wait()
        sb = pltpu.async_copy(rb, acc_s.at[ib], ssb, add=True)
        sa.wait()
        sb.wait()
        return carry

    lax.fori_loop(0, _SCH // 2, pair, 0)
    plsc.subcore_barrier()

    def ostep(k, carry):
        r0 = s * _RPT + k * _CH
        pltpu.sync_copy(acc_s.at[pl.ds(r0, _CH)], ra)
        pltpu.sync_copy(ra, acc_hbm.at[pl.ds(c * _NPAD + r0, _CH)])
        return carry

    lax.fori_loop(0, _RPT // _CH, ostep, 0)


_sc_scatter = pl.kernel(
    _sc_scatter_body,
    out_type=jax.ShapeDtypeStruct((_NC * _NPAD, _D), _F32),
    mesh=_sc_mesh,
    scratch_types=[
        pltpu.VMEM((_CH,), jnp.int32),
        pltpu.VMEM((_CH,), jnp.int32),
        pltpu.VMEM((_CH, _C), _F32),
        pltpu.VMEM((_CH, _C), _F32),
        pltpu.VMEM((_CH, _D), _F32),
        pltpu.VMEM((_CH, _D), _F32),
        pltpu.VMEM((_ZCH, _D), _F32),
        pltpu.VMEM_SHARED((_NPAD, _D), _F32),
    ] + [pltpu.SemaphoreType.DMA] * 6,
)


def _final_body(acc_ref, sk_ref, out_ref):
    am = acc_ref[0]
    dd = acc_ref[1]
    out_ref[...] = am / (dd + 1e-16) + sk_ref[...]


_final = pl.pallas_call(
    _final_body,
    grid=(_N // _BN,),
    in_specs=[
        pl.BlockSpec((_NC, _BN, _D), lambda i: (0, i, 0)),
        pl.BlockSpec((_BN, _D), lambda i: (i, 0)),
    ],
    out_specs=pl.BlockSpec((_BN, _D), lambda i: (i, 0)),
    out_shape=jax.ShapeDtypeStruct((_N, _D), _F32),
)


def kernel(x, edge_index, edge_attr, Wq, bq, Wk, bk, Wv, bv, We, Wskip, bskip):
    src = edge_index[0].astype(jnp.int32)
    dst = edge_index[1].astype(jnp.int32)
    wkv = jnp.concatenate([Wk, Wv], axis=1)
    bkv = jnp.concatenate([bk, bv], axis=0)
    qn, kv, sk = _proj(x, Wq, bq.reshape(1, _D), wkv, bkv.reshape(1, 2 * _D),
                       Wskip, bskip.reshape(1, _D))
    dst4 = dst.reshape(_NW, _GCH, _CH)
    src4 = src.reshape(_NW, _GCH, _CH)
    qd, kvs = _sc_gather(qn, kv, dst4, src4)
    m, exb = _edges(edge_attr, qd, kvs, We)
    z = jnp.zeros((_ZCH, _D), _F32)
    acc = _sc_scatter(m, exb, dst, z)
    return _final(acc.reshape(_NC, _NPAD, _D), sk)


# speedup vs baseline: 1.0607x; 1.0607x over previous
import jax
import jax.numpy as jnp
from jax import lax
from jax.experimental import pallas as pl
from jax.experimental.pallas import tpu as pltpu
from jax.experimental.pallas import tpu_sc as plsc

_N = 10000
_E = 320000
_D = 128
_H = 8
_C = 16
_NC = 2
_NS = 16
_NW = _NC * _NS
_PERW = _E // _NW
_EPT = _E // _NS
_CH = 80
_GCH = _PERW // _CH
_SCH = _EPT // _CH
_NPAD = 10240
_RPT = _NPAD // _NS
_ZCH = 64

_BN = 1000
_BE = 2000

_F32 = jnp.float32

_sc_mesh = plsc.VectorSubcoreMesh(
    core_axis_name="c", subcore_axis_name="s", num_cores=_NC, num_subcores=_NS)


def _head_indicator(rows, cols, row_div, col_div):
    r = lax.broadcasted_iota(jnp.int32, (rows, cols), 0) // row_div
    c = lax.broadcasted_iota(jnp.int32, (rows, cols), 1) // col_div
    return (r == c).astype(_F32)


def _proj_body(x_ref, wq_ref, bq_ref, wkv_ref, bkv_ref, ws_ref, bs_ref,
               qn_ref, kv_ref, sk_ref):
    xb = x_ref[...]
    qn_ref[...] = jnp.dot(xb, wq_ref[...], preferred_element_type=_F32) + bq_ref[...]
    kv_ref[...] = jnp.dot(xb, wkv_ref[...], preferred_element_type=_F32) + bkv_ref[...]
    sk_ref[...] = jnp.dot(xb, ws_ref[...], preferred_element_type=_F32) + bs_ref[...]


_proj = pl.pallas_call(
    _proj_body,
    grid=(_N // _BN,),
    in_specs=[
        pl.BlockSpec((_BN, _D), lambda i: (i, 0)),
        pl.BlockSpec((_D, _D), lambda i: (0, 0)),
        pl.BlockSpec((1, _D), lambda i: (0, 0)),
        pl.BlockSpec((_D, 2 * _D), lambda i: (0, 0)),
        pl.BlockSpec((1, 2 * _D), lambda i: (0, 0)),
        pl.BlockSpec((_D, _D), lambda i: (0, 0)),
        pl.BlockSpec((1, _D), lambda i: (0, 0)),
    ],
    out_specs=[
        pl.BlockSpec((_BN, _D), lambda i: (i, 0)),
        pl.BlockSpec((_BN, 2 * _D), lambda i: (i, 0)),
        pl.BlockSpec((_BN, _D), lambda i: (i, 0)),
    ],
    out_shape=[
        jax.ShapeDtypeStruct((_N, _D), _F32),
        jax.ShapeDtypeStruct((_N, 2 * _D), _F32),
        jax.ShapeDtypeStruct((_N, _D), _F32),
    ],
)


def _sc_gather_body(qn_hbm, kv_hbm, dst_hbm, src_hbm, qd_hbm, kvs_hbm,
                    dstv, srcv, qa, kva, qb, kvb, qc, kvc,
                    gqa, gkva, gqb, gkvb, gqc, gkvc,
                    wqa, wkva, wqb, wkvb, wqc, wkvc):
    wid = lax.axis_index("s") * _NC + lax.axis_index("c")
    base = wid * _PERW
    pltpu.sync_copy(dst_hbm.at[wid], dstv)
    pltpu.sync_copy(src_hbm.at[wid], srcv)

    def triple(t, carry):
        ja = 3 * t
        jb = 3 * t + 1
        jc = 3 * t + 2
        offa = base + ja * _CH
        offb = base + jb * _CH
        offc = base + jc * _CH
        ga1 = pltpu.async_copy(qn_hbm.at[dstv.at[ja]], qa, gqa)
        ga2 = pltpu.async_copy(kv_hbm.at[srcv.at[ja]], kva, gkva)
        gb1 = pltpu.async_copy(qn_hbm.at[dstv.at[jb]], qb, gqb)
        gb2 = pltpu.async_copy(kv_hbm.at[srcv.at[jb]], kvb, gkvb)
        gc1 = pltpu.async_copy(qn_hbm.at[dstv.at[jc]], qc, gqc)
        gc2 = pltpu.async_copy(kv_hbm.at[srcv.at[jc]], kvc, gkvc)
        ga1.wait()
        ga2.wait()
        wa1 = pltpu.async_copy(qa, qd_hbm.at[pl.ds(offa, _CH)], wqa)
        wa2 = pltpu.async_copy(kva, kvs_hbm.at[pl.ds(offa, _CH)], wkva)
        gb1.wait()
        gb2.wait()
        wb1 = pltpu.async_copy(qb, qd_hbm.at[pl.ds(offb, _CH)], wqb)
        wb2 = pltpu.async_copy(kvb, kvs_hbm.at[pl.ds(offb, _CH)], wkvb)
        gc1.wait()
        gc2.wait()
        wc1 = pltpu.async_copy(qc, qd_hbm.at[pl.ds(offc, _CH)], wqc)
        wc2 = pltpu.async_copy(kvc, kvs_hbm.at[pl.ds(offc, _CH)], wkvc)
        wa1.wait()
        wa2.wait()
        wb1.wait()
        wb2.wait()
        wc1.wait()
        wc2.wait()
        return carry

    lax.fori_loop(0, _GCH // 3, triple, 0)

    ja = _GCH - 2
    jb = _GCH - 1
    offa = base + ja * _CH
    offb = base + jb * _CH
    ga1 = pltpu.async_copy(qn_hbm.at[dstv.at[ja]], qa, gqa)
    ga2 = pltpu.async_copy(kv_hbm.at[srcv.at[ja]], kva, gkva)
    gb1 = pltpu.async_copy(qn_hbm.at[dstv.at[jb]], qb, gqb)
    gb2 = pltpu.async_copy(kv_hbm.at[srcv.at[jb]], kvb, gkvb)
    ga1.wait()
    ga2.wait()
    wa1 = pltpu.async_copy(qa, qd_hbm.at[pl.ds(offa, _CH)], wqa)
    wa2 = pltpu.async_copy(kva, kvs_hbm.at[pl.ds(offa, _CH)], wkva)
    gb1.wait()
    gb2.wait()
    pltpu.sync_copy(qb, qd_hbm.at[pl.ds(offb, _CH)])
    pltpu.sync_copy(kvb, kvs_hbm.at[pl.ds(offb, _CH)])
    wa1.wait()
    wa2.wait()


_sc_gather = pl.kernel(
    _sc_gather_body,
    out_type=(
        jax.ShapeDtypeStruct((_E, _D), _F32),
        jax.ShapeDtypeStruct((_E, 2 * _D), _F32),
    ),
    mesh=_sc_mesh,
    scratch_types=[
        pltpu.VMEM((_GCH, _CH), jnp.int32),
        pltpu.VMEM((_GCH, _CH), jnp.int32),
        pltpu.VMEM((_CH, _D), _F32),
        pltpu.VMEM((_CH, 2 * _D), _F32),
        pltpu.VMEM((_CH, _D), _F32),
        pltpu.VMEM((_CH, 2 * _D), _F32),
        pltpu.VMEM((_CH, _D), _F32),
        pltpu.VMEM((_CH, 2 * _D), _F32),
    ] + [pltpu.SemaphoreType.DMA] * 12,
)


def _edge_body(ea_ref, qd_ref, kvs_ref, we_ref, m_ref, ex_ref):
    e = jnp.dot(ea_ref[...], we_ref[...], preferred_element_type=_F32)
    kk = kvs_ref[:, :_D] + e
    vv = kvs_ref[:, _D:] + e
    s_fold = _head_indicator(_D, _H, _C, 1)
    s_bcast = _head_indicator(_H, _D, 1, _C)
    alpha = jnp.dot(qd_ref[...] * kk, s_fold, preferred_element_type=_F32) * 0.25
    exv = jnp.exp(alpha)
    exb = jnp.dot(exv, s_bcast, preferred_element_type=_F32)
    m_ref[...] = vv * exb
    ex_ref[...] = exb


_edges = pl.pallas_call(
    _edge_body,
    grid=(_E // _BE,),
    in_specs=[
        pl.BlockSpec((_BE, _D), lambda i: (i, 0)),
        pl.BlockSpec((_BE, _D), lambda i: (i, 0)),
        pl.BlockSpec((_BE, 2 * _D), lambda i: (i, 0)),
        pl.BlockSpec((_D, _D), lambda i: (0, 0)),
    ],
    out_specs=[
        pl.BlockSpec((_BE, _D), lambda i: (i, 0)),
        pl.BlockSpec((_BE, _D), lambda i: (i, 0)),
    ],
    out_shape=[
        jax.ShapeDtypeStruct((_E, _D), _F32),
        jax.ShapeDtypeStruct((_E, _D), _F32),
    ],
)


def _sc_scatter_body(m_hbm, ex_hbm, dst_hbm, z_hbm, acc_hbm,
                     ia, ib, ra, rb, zb, acc_s,
                     sia, sib, sla, slb, ssa, ssb):
    c = lax.axis_index("c")
    s = lax.axis_index("s")
    base = s * _EPT

    pltpu.sync_copy(z_hbm, zb)

    def zstep(k, carry):
        r0 = s * _RPT + k * _ZCH
        pltpu.async_copy(zb, acc_s.at[pl.ds(r0, _ZCH)], sia)
        return carry

    lax.fori_loop(0, _RPT // _ZCH, zstep, 0)

    def zdrain(k, carry):
        pltpu.make_async_copy(zb, acc_s.at[pl.ds(s * _RPT, _ZCH)], sia).wait()
        return carry

    lax.fori_loop(0, _RPT // _ZCH, zdrain, 0)
    plsc.subcore_barrier()

    def pair(t, carry):
        offa = base + (2 * t) * _CH
        offb = base + (2 * t + 1) * _CH
        cia = pltpu.async_copy(dst_hbm.at[pl.ds(offa, _CH)], ia, sia)
        cib = pltpu.async_copy(dst_hbm.at[pl.ds(offb, _CH)], ib, sib)

        @pl.when(c == 0)
        def _load_m():
            pltpu.async_copy(m_hbm.at[pl.ds(offa, _CH)], ra, sla)
            pltpu.async_copy(m_hbm.at[pl.ds(offb, _CH)], rb, slb)

        @pl.when(c == 1)
        def _load_ex():
            pltpu.async_copy(ex_hbm.at[pl.ds(offa, _CH)], ra, sla)
            pltpu.async_copy(ex_hbm.at[pl.ds(offb, _CH)], rb, slb)

        la = pltpu.make_async_copy(m_hbm.at[pl.ds(offa, _CH)], ra, sla)
        lb = pltpu.make_async_copy(m_hbm.at[pl.ds(offb, _CH)], rb, slb)
        cia.wait()
        la.wait()
        sa = pltpu.async_copy(ra, acc_s.at[ia], ssa, add=True)
        cib.wait()
        lb.wait()
        sb = pltpu.async_copy(rb, acc_s.at[ib], ssb, add=True)
        sa.wait()
        sb.wait()
        return carry

    lax.fori_loop(0, _SCH // 2, pair, 0)
    plsc.subcore_barrier()

    def opair(k, carry):
        r0 = s * _RPT + (2 * k) * _CH
        r1 = s * _RPT + (2 * k + 1) * _CH
        oa = pltpu.async_copy(acc_s.at[pl.ds(r0, _CH)], ra, sla)
        ob = pltpu.async_copy(acc_s.at[pl.ds(r1, _CH)], rb, slb)
        oa.wait()
        wa = pltpu.async_copy(ra, acc_hbm.at[pl.ds(c * _NPAD + r0, _CH)], ssa)
        ob.wait()
        wb = pltpu.async_copy(rb, acc_hbm.at[pl.ds(c * _NPAD + r1, _CH)], ssb)
        wa.wait()
        wb.wait()
        return carry

    lax.fori_loop(0, _RPT // _CH // 2, opair, 0)


_sc_scatter = pl.kernel(
    _sc_scatter_body,
    out_type=jax.ShapeDtypeStruct((_NC * _NPAD, _D), _F32),
    mesh=_sc_mesh,
    scratch_types=[
        pltpu.VMEM((_CH,), jnp.int32),
        pltpu.VMEM((_CH,), jnp.int32),
        pltpu.VMEM((_CH, _D), _F32),
        pltpu.VMEM((_CH, _D), _F32),
        pltpu.VMEM((_ZCH, _D), _F32),
        pltpu.VMEM_SHARED((_NPAD, _D), _F32),
    ] + [pltpu.SemaphoreType.DMA] * 6,
)


def _final_body(acc_ref, sk_ref, out_ref):
    am = acc_ref[0]
    dd = acc_ref[1]
    out_ref[...] = am / (dd + 1e-16) + sk_ref[...]


_final = pl.pallas_call(
    _final_body,
    grid=(_N // _BN,),
    in_specs=[
        pl.BlockSpec((_NC, _BN, _D), lambda i: (0, i, 0)),
        pl.BlockSpec((_BN, _D), lambda i: (i, 0)),
    ],
    out_specs=pl.BlockSpec((_BN, _D), lambda i: (i, 0)),
    out_shape=jax.ShapeDtypeStruct((_N, _D), _F32),
)


def kernel(x, edge_index, edge_attr, Wq, bq, Wk, bk, Wv, bv, We, Wskip, bskip):
    src = edge_index[0].astype(jnp.int32)
    dst = edge_index[1].astype(jnp.int32)
    wkv = jnp.concatenate([Wk, Wv], axis=1)
    bkv = jnp.concatenate([bk, bv], axis=0)
    qn, kv, sk = _proj(x, Wq, bq.reshape(1, _D), wkv, bkv.reshape(1, 2 * _D),
                       Wskip, bskip.reshape(1, _D))
    dst4 = dst.reshape(_NW, _GCH, _CH)
    src4 = src.reshape(_NW, _GCH, _CH)
    qd, kvs = _sc_gather(qn, kv, dst4, src4)
    m, exb = _edges(edge_attr, qd, kvs, We)
    z = jnp.zeros((_ZCH, _D), _F32)
    acc = _sc_scatter(m, exb, dst, z)
    return _final(acc.reshape(_NC, _NPAD, _D), sk)


# speedup vs baseline: 1.0796x; 1.0178x over previous
import jax
import jax.numpy as jnp
from jax import lax
from jax.experimental import pallas as pl
from jax.experimental.pallas import tpu as pltpu
from jax.experimental.pallas import tpu_sc as plsc

_N = 10000
_E = 320000
_D = 128
_H = 8
_C = 16
_NC = 2
_NS = 16
_NW = _NC * _NS
_PERW = _E // _NW
_EPT = _E // _NS
_CH = 80
_GCH = _PERW // _CH
_SCH = _EPT // _CH
_NPAD = 10240
_RPT = _NPAD // _NS
_ZCH = 64

_BN = 1000
_BE = 4000

_F32 = jnp.float32

_sc_mesh = plsc.VectorSubcoreMesh(
    core_axis_name="c", subcore_axis_name="s", num_cores=_NC, num_subcores=_NS)


def _head_indicator(rows, cols, row_div, col_div):
    r = lax.broadcasted_iota(jnp.int32, (rows, cols), 0) // row_div
    c = lax.broadcasted_iota(jnp.int32, (rows, cols), 1) // col_div
    return (r == c).astype(_F32)


def _proj_body(x_ref, wq_ref, bq_ref, wkv_ref, bkv_ref, ws_ref, bs_ref,
               qn_ref, kv_ref, sk_ref):
    xb = x_ref[...]
    qn_ref[...] = jnp.dot(xb, wq_ref[...], preferred_element_type=_F32) + bq_ref[...]
    kv_ref[...] = jnp.dot(xb, wkv_ref[...], preferred_element_type=_F32) + bkv_ref[...]
    sk_ref[...] = jnp.dot(xb, ws_ref[...], preferred_element_type=_F32) + bs_ref[...]


_proj = pl.pallas_call(
    _proj_body,
    grid=(_N // _BN,),
    in_specs=[
        pl.BlockSpec((_BN, _D), lambda i: (i, 0)),
        pl.BlockSpec((_D, _D), lambda i: (0, 0)),
        pl.BlockSpec((1, _D), lambda i: (0, 0)),
        pl.BlockSpec((_D, 2 * _D), lambda i: (0, 0)),
        pl.BlockSpec((1, 2 * _D), lambda i: (0, 0)),
        pl.BlockSpec((_D, _D), lambda i: (0, 0)),
        pl.BlockSpec((1, _D), lambda i: (0, 0)),
    ],
    out_specs=[
        pl.BlockSpec((_BN, _D), lambda i: (i, 0)),
        pl.BlockSpec((_BN, 2 * _D), lambda i: (i, 0)),
        pl.BlockSpec((_BN, _D), lambda i: (i, 0)),
    ],
    out_shape=[
        jax.ShapeDtypeStruct((_N, _D), _F32),
        jax.ShapeDtypeStruct((_N, 2 * _D), _F32),
        jax.ShapeDtypeStruct((_N, _D), _F32),
    ],
)


def _sc_gather_body(qn_hbm, kv_hbm, dst_hbm, src_hbm, qd_hbm, kvs_hbm,
                    dstv, srcv, qa, kva, qb, kvb, qc, kvc,
                    gqa, gkva, gqb, gkvb, gqc, gkvc,
                    wqa, wkva, wqb, wkvb, wqc, wkvc):
    wid = lax.axis_index("s") * _NC + lax.axis_index("c")
    base = wid * _PERW
    pltpu.sync_copy(dst_hbm.at[wid], dstv)
    pltpu.sync_copy(src_hbm.at[wid], srcv)

    def triple(t, carry):
        ja = 3 * t
        jb = 3 * t + 1
        jc = 3 * t + 2
        offa = base + ja * _CH
        offb = base + jb * _CH
        offc = base + jc * _CH
        ga1 = pltpu.async_copy(qn_hbm.at[dstv.at[ja]], qa, gqa)
        ga2 = pltpu.async_copy(kv_hbm.at[srcv.at[ja]], kva, gkva)
        gb1 = pltpu.async_copy(qn_hbm.at[dstv.at[jb]], qb, gqb)
        gb2 = pltpu.async_copy(kv_hbm.at[srcv.at[jb]], kvb, gkvb)
        gc1 = pltpu.async_copy(qn_hbm.at[dstv.at[jc]], qc, gqc)
        gc2 = pltpu.async_copy(kv_hbm.at[srcv.at[jc]], kvc, gkvc)
        ga1.wait()
        ga2.wait()
        wa1 = pltpu.async_copy(qa, qd_hbm.at[pl.ds(offa, _CH)], wqa)
        wa2 = pltpu.async_copy(kva, kvs_hbm.at[pl.ds(offa, _CH)], wkva)
        gb1.wait()
        gb2.wait()
        wb1 = pltpu.async_copy(qb, qd_hbm.at[pl.ds(offb, _CH)], wqb)
        wb2 = pltpu.async_copy(kvb, kvs_hbm.at[pl.ds(offb, _CH)], wkvb)
        gc1.wait()
        gc2.wait()
        wc1 = pltpu.async_copy(qc, qd_hbm.at[pl.ds(offc, _CH)], wqc)
        wc2 = pltpu.async_copy(kvc, kvs_hbm.at[pl.ds(offc, _CH)], wkvc)
        wa1.wait()
        wa2.wait()
        wb1.wait()
        wb2.wait()
        wc1.wait()
        wc2.wait()
        return carry

    lax.fori_loop(0, _GCH // 3, triple, 0)

    ja = _GCH - 2
    jb = _GCH - 1
    offa = base + ja * _CH
    offb = base + jb * _CH
    ga1 = pltpu.async_copy(qn_hbm.at[dstv.at[ja]], qa, gqa)
    ga2 = pltpu.async_copy(kv_hbm.at[srcv.at[ja]], kva, gkva)
    gb1 = pltpu.async_copy(qn_hbm.at[dstv.at[jb]], qb, gqb)
    gb2 = pltpu.async_copy(kv_hbm.at[srcv.at[jb]], kvb, gkvb)
    ga1.wait()
    ga2.wait()
    wa1 = pltpu.async_copy(qa, qd_hbm.at[pl.ds(offa, _CH)], wqa)
    wa2 = pltpu.async_copy(kva, kvs_hbm.at[pl.ds(offa, _CH)], wkva)
    gb1.wait()
    gb2.wait()
    pltpu.sync_copy(qb, qd_hbm.at[pl.ds(offb, _CH)])
    pltpu.sync_copy(kvb, kvs_hbm.at[pl.ds(offb, _CH)])
    wa1.wait()
    wa2.wait()


_sc_gather = pl.kernel(
    _sc_gather_body,
    out_type=(
        jax.ShapeDtypeStruct((_E, _D), _F32),
        jax.ShapeDtypeStruct((_E, 2 * _D), _F32),
    ),
    mesh=_sc_mesh,
    scratch_types=[
        pltpu.VMEM((_GCH, _CH), jnp.int32),
        pltpu.VMEM((_GCH, _CH), jnp.int32),
        pltpu.VMEM((_CH, _D), _F32),
        pltpu.VMEM((_CH, 2 * _D), _F32),
        pltpu.VMEM((_CH, _D), _F32),
        pltpu.VMEM((_CH, 2 * _D), _F32),
        pltpu.VMEM((_CH, _D), _F32),
        pltpu.VMEM((_CH, 2 * _D), _F32),
    ] + [pltpu.SemaphoreType.DMA] * 12,
)


def _edge_body(ea_ref, qd_ref, kvs_ref, we_ref, m_ref, ex_ref):
    e = jnp.dot(ea_ref[...], we_ref[...], preferred_element_type=_F32)
    kk = kvs_ref[:, :_D] + e
    vv = kvs_ref[:, _D:] + e
    s_fold = _head_indicator(_D, _H, _C, 1)
    s_bcast = _head_indicator(_H, _D, 1, _C)
    alpha = jnp.dot(qd_ref[...] * kk, s_fold, preferred_element_type=_F32) * 0.25
    exv = jnp.exp(alpha)
    exb = jnp.dot(exv, s_bcast, preferred_element_type=_F32)
    m_ref[...] = vv * exb
    ex_ref[...] = exb


_edges = pl.pallas_call(
    _edge_body,
    grid=(_E // _BE,),
    in_specs=[
        pl.BlockSpec((_BE, _D), lambda i: (i, 0)),
        pl.BlockSpec((_BE, _D), lambda i: (i, 0)),
        pl.BlockSpec((_BE, 2 * _D), lambda i: (i, 0)),
        pl.BlockSpec((_D, _D), lambda i: (0, 0)),
    ],
    out_specs=[
        pl.BlockSpec((_BE, _D), lambda i: (i, 0)),
        pl.BlockSpec((_BE, _D), lambda i: (i, 0)),
    ],
    out_shape=[
        jax.ShapeDtypeStruct((_E, _D), _F32),
        jax.ShapeDtypeStruct((_E, _D), _F32),
    ],
)


def _sc_scatter_body(m_hbm, ex_hbm, dst_hbm, z_hbm, acc_hbm,
                     ia, ib, ra, rb, zb, acc_s,
                     sia, sib, sla, slb, ssa, ssb):
    c = lax.axis_index("c")
    s = lax.axis_index("s")
    base = s * _EPT

    pltpu.sync_copy(z_hbm, zb)

    def zstep(k, carry):
        r0 = s * _RPT + k * _ZCH
        pltpu.async_copy(zb, acc_s.at[pl.ds(r0, _ZCH)], sia)
        return carry

    lax.fori_loop(0, _RPT // _ZCH, zstep, 0)

    def zdrain(k, carry):
        pltpu.make_async_copy(zb, acc_s.at[pl.ds(s * _RPT, _ZCH)], sia).wait()
        return carry

    lax.fori_loop(0, _RPT // _ZCH, zdrain, 0)
    plsc.subcore_barrier()

    def pair(t, carry):
        offa = base + (2 * t) * _CH
        offb = base + (2 * t + 1) * _CH
        cia = pltpu.async_copy(dst_hbm.at[pl.ds(offa, _CH)], ia, sia)
        cib = pltpu.async_copy(dst_hbm.at[pl.ds(offb, _CH)], ib, sib)

        @pl.when(c == 0)
        def _load_m():
            pltpu.async_copy(m_hbm.at[pl.ds(offa, _CH)], ra, sla)
            pltpu.async_copy(m_hbm.at[pl.ds(offb, _CH)], rb, slb)

        @pl.when(c == 1)
        def _load_ex():
            pltpu.async_copy(ex_hbm.at[pl.ds(offa, _CH)], ra, sla)
            pltpu.async_copy(ex_hbm.at[pl.ds(offb, _CH)], rb, slb)

        la = pltpu.make_async_copy(m_hbm.at[pl.ds(offa, _CH)], ra, sla)
        lb = pltpu.make_async_copy(m_hbm.at[pl.ds(offb, _CH)], rb, slb)
        cia.wait()
        la.wait()
        sa = pltpu.async_copy(ra, acc_s.at[ia], ssa, add=True)
        cib.wait()
        lb.wait()
        sb = pltpu.async_copy(rb, acc_s.at[ib], ssb, add=True)
        sa.wait()
        sb.wait()
        return carry

    lax.fori_loop(0, _SCH // 2, pair, 0)
    plsc.subcore_barrier()

    def opair(k, carry):
        r0 = s * _RPT + (2 * k) * _CH
        r1 = s * _RPT + (2 * k + 1) * _CH
        oa = pltpu.async_copy(acc_s.at[pl.ds(r0, _CH)], ra, sla)
        ob = pltpu.async_copy(acc_s.at[pl.ds(r1, _CH)], rb, slb)
        oa.wait()
        wa = pltpu.async_copy(ra, acc_hbm.at[pl.ds(c * _NPAD + r0, _CH)], ssa)
        ob.wait()
        wb = pltpu.async_copy(rb, acc_hbm.at[pl.ds(c * _NPAD + r1, _CH)], ssb)
        wa.wait()
        wb.wait()
        return carry

    lax.fori_loop(0, _RPT // _CH // 2, opair, 0)


_sc_scatter = pl.kernel(
    _sc_scatter_body,
    out_type=jax.ShapeDtypeStruct((_NC * _NPAD, _D), _F32),
    mesh=_sc_mesh,
    scratch_types=[
        pltpu.VMEM((_CH,), jnp.int32),
        pltpu.VMEM((_CH,), jnp.int32),
        pltpu.VMEM((_CH, _D), _F32),
        pltpu.VMEM((_CH, _D), _F32),
        pltpu.VMEM((_ZCH, _D), _F32),
        pltpu.VMEM_SHARED((_NPAD, _D), _F32),
    ] + [pltpu.SemaphoreType.DMA] * 6,
)


def _final_body(acc_ref, sk_ref, out_ref):
    am = acc_ref[0]
    dd = acc_ref[1]
    out_ref[...] = am / (dd + 1e-16) + sk_ref[...]


_final = pl.pallas_call(
    _final_body,
    grid=(_N // _BN,),
    in_specs=[
        pl.BlockSpec((_NC, _BN, _D), lambda i: (0, i, 0)),
        pl.BlockSpec((_BN, _D), lambda i: (i, 0)),
    ],
    out_specs=pl.BlockSpec((_BN, _D), lambda i: (i, 0)),
    out_shape=jax.ShapeDtypeStruct((_N, _D), _F32),
)


def kernel(x, edge_index, edge_attr, Wq, bq, Wk, bk, Wv, bv, We, Wskip, bskip):
    src = edge_index[0].astype(jnp.int32)
    dst = edge_index[1].astype(jnp.int32)
    wkv = jnp.concatenate([Wk, Wv], axis=1)
    bkv = jnp.concatenate([bk, bv], axis=0)
    qn, kv, sk = _proj(x, Wq, bq.reshape(1, _D), wkv, bkv.reshape(1, 2 * _D),
                       Wskip, bskip.reshape(1, _D))
    dst4 = dst.reshape(_NW, _GCH, _CH)
    src4 = src.reshape(_NW, _GCH, _CH)
    qd, kvs = _sc_gather(qn, kv, dst4, src4)
    m, exb = _edges(edge_attr, qd, kvs, We)
    z = jnp.zeros((_ZCH, _D), _F32)
    acc = _sc_scatter(m, exb, dst, z)
    return _final(acc.reshape(_NC, _NPAD, _D), sk)
